# baseline (speedup 1.0000x reference)
import functools

import jax
import jax.numpy as jnp
from jax import lax
from jax.experimental import pallas as pl
from jax.experimental.pallas import tpu as pltpu
from jax.experimental.pallas import tpu_sc as plsc

N, E, T = 10000, 160000, 320000
ATOM, BOND, ANGLE = 128, 128, 16
C2 = 2 * BOND

_SC_INFO = plsc.get_sparse_core_info()
_NC = _SC_INFO.num_cores
_NS = _SC_INFO.num_subcores
NW = _NC * _NS


_GB = 80
_CHUNK = T // NW
_NBLK = _CHUNK // _GB


_CW = C2 // 2


def _bf2(v):
    return plsc.bitcast(v, jnp.bfloat16)


def _gather_sum_body(pa_hbm, pj_hbm, pi_hbm, pk_hbm, j_hbm, i_hbm, k_hbm,
                     x_hbm, jb, ib, kb, xa, gj, gi, gk,
                     sem_a, sem_j, sem_i, sem_k):
    wid = lax.axis_index("s") * _NC + lax.axis_index("c")

    def blk_body(b, carry):
        base = wid * _CHUNK + b * _GB
        pltpu.sync_copy(j_hbm.at[pl.ds(base, _GB)], jb)
        pltpu.sync_copy(i_hbm.at[pl.ds(base, _GB)], ib)
        pltpu.sync_copy(k_hbm.at[pl.ds(base, _GB)], kb)
        ca = pltpu.async_copy(pa_hbm.at[pl.ds(base, _GB)], xa, sem_a)
        cj = pltpu.async_copy(pj_hbm.at[jb], gj, sem_j)
        ci = pltpu.async_copy(pi_hbm.at[ib], gi, sem_i)
        ck = pltpu.async_copy(pk_hbm.at[kb], gk, sem_k)
        ca.wait()
        cj.wait()
        ci.wait()
        ck.wait()

        def row_body(r, c2):
            for c in range(_CW // 16):
                sl = pl.ds(c * 16, 16)
                s = (_bf2(xa[r, sl]) + _bf2(gj[r, sl])) +                     (_bf2(gi[r, sl]) + _bf2(gk[r, sl]))
                xa[r, sl] = plsc.bitcast(s, jnp.int32)
            return c2

        lax.fori_loop(0, _GB, row_body, 0)
        pltpu.sync_copy(xa, x_hbm.at[pl.ds(base, _GB)])
        return carry

    lax.fori_loop(0, _NBLK, blk_body, 0)


def _gather_sum(Pa, Pj, Pi, Pk, j_idx, i_idx, k_idx):
    mesh = plsc.VectorSubcoreMesh(core_axis_name="c", subcore_axis_name="s")
    f = functools.partial(
        pl.kernel,
        mesh=mesh,
        compiler_params=pltpu.CompilerParams(needs_layout_passes=False),
        out_type=jax.ShapeDtypeStruct((T, _CW), jnp.int32),
        scratch_types=[
            pltpu.VMEM((_GB,), jnp.int32),
            pltpu.VMEM((_GB,), jnp.int32),
            pltpu.VMEM((_GB,), jnp.int32),
            pltpu.VMEM((_GB, _CW), jnp.int32),
            pltpu.VMEM((_GB, _CW), jnp.int32),
            pltpu.VMEM((_GB, _CW), jnp.int32),
            pltpu.VMEM((_GB, _CW), jnp.int32),
            pltpu.SemaphoreType.DMA,
            pltpu.SemaphoreType.DMA,
            pltpu.SemaphoreType.DMA,
            pltpu.SemaphoreType.DMA,
        ],
    )(_gather_sum_body)
    return f(Pa, Pj, Pi, Pk, j_idx, i_idx, k_idx)


def _mm_body(x_ref, w_ref, o_ref):
    o_ref[...] = jnp.dot(x_ref[...], w_ref[...],
                         preferred_element_type=jnp.float32).astype(o_ref.dtype)


def _rowmm(x, w, blk, out_dtype=jnp.float32):
    m, k = x.shape
    n = w.shape[1]
    return pl.pallas_call(
        _mm_body,
        grid=(m // blk,),
        in_specs=[pl.BlockSpec((blk, k), lambda i: (i, 0)),
                  pl.BlockSpec((k, n), lambda i: (0, 0))],
        out_specs=pl.BlockSpec((blk, n), lambda i: (i, 0)),
        out_shape=jax.ShapeDtypeStruct((m, n), out_dtype),
    )(x, w)


def _stats_body(x_ref, o_ref):
    @pl.when(pl.program_id(0) == 0)
    def _():
        o_ref[...] = jnp.zeros_like(o_ref)
    x = x_ref[...].astype(jnp.float32)
    s = jnp.sum(x, axis=0)
    sq = jnp.sum(x * x, axis=0)
    o_ref[0, :] += s
    o_ref[1, :] += sq


def _stats(x, blk):
    m, n = x.shape
    return pl.pallas_call(
        _stats_body,
        grid=(m // blk,),
        in_specs=[pl.BlockSpec((blk, n), lambda i: (i, 0))],
        out_specs=pl.BlockSpec((8, n), lambda i: (0, 0)),
        out_shape=jax.ShapeDtypeStruct((8, n), jnp.float32),
    )(x)


def _act_body(x_ref, st_ref, p_ref, o_ref):
    x = x_ref[...].astype(jnp.float32)
    s = st_ref[0, :]
    sq = st_ref[1, :]
    mean = s / T
    var = sq / T - mean * mean
    inv = jax.lax.rsqrt(var + 1e-5)
    gamma = jnp.concatenate([p_ref[0, :], p_ref[2, :]])
    beta = jnp.concatenate([p_ref[1, :], p_ref[3, :]])
    y = (x - mean) * inv * gamma + beta
    core = y[:, :BOND]
    gate = y[:, BOND:]
    core = core * jax.nn.sigmoid(core)
    gate = jax.nn.sigmoid(gate)
    o_ref[...] = (core * gate).astype(o_ref.dtype)


def _activate(x, stats, params, blk):
    m = x.shape[0]
    return pl.pallas_call(
        _act_body,
        grid=(m // blk,),
        in_specs=[pl.BlockSpec((blk, C2), lambda i: (i, 0)),
                  pl.BlockSpec((8, C2), lambda i: (0, 0)),
                  pl.BlockSpec((8, BOND), lambda i: (0, 0))],
        out_specs=pl.BlockSpec((blk, BOND), lambda i: (i, 0)),
        out_shape=jax.ShapeDtypeStruct((m, BOND), jnp.bfloat16),
    )(x, stats, params)


def _final_body(seg_ref, edge_ref, w_ref, out_ref):
    out_ref[...] = jnp.dot(seg_ref[...], w_ref[...],
                           preferred_element_type=jnp.float32) + edge_ref[...]


def _final_matmul(segsum, edge_feat, W_out):
    BLK = 1600
    return pl.pallas_call(
        _final_body,
        grid=(E // BLK,),
        in_specs=[
            pl.BlockSpec((BLK, BOND), lambda i: (i, 0)),
            pl.BlockSpec((BLK, BOND), lambda i: (i, 0)),
            pl.BlockSpec((BOND, BOND), lambda i: (0, 0)),
        ],
        out_specs=pl.BlockSpec((BLK, BOND), lambda i: (i, 0)),
        out_shape=jax.ShapeDtypeStruct((E, BOND), jnp.float32),
    )(segsum, edge_feat, W_out)


def kernel(vertex_feat, edge_feat, angle_feat, edge_index, k_idx, j_idx, i_idx,
           W_core_src, W_core_dst, W_core_bond, W_core_angle,
           W_gate_src, W_gate_dst, W_gate_bond, W_gate_angle,
           bn_core_gamma, bn_core_beta, bn_gate_gamma, bn_gate_beta, W_out):
    k_idx = k_idx.astype(jnp.int32)
    j_idx = j_idx.astype(jnp.int32)
    i_idx = i_idx.astype(jnp.int32)

    Wj = jnp.concatenate([W_core_src, W_gate_src], axis=1)
    Wi = jnp.concatenate([W_core_dst, W_gate_dst], axis=1)
    Wk = jnp.concatenate([W_core_bond, W_gate_bond], axis=1)
    Wa = jnp.concatenate([W_core_angle, W_gate_angle], axis=1)
    bf16 = jnp.bfloat16

    def _as_i32(a):
        m = a.shape[0]
        return jax.lax.bitcast_convert_type(a.reshape(m, _CW, 2), jnp.int32)

    Pj = _as_i32(_rowmm(vertex_feat, Wj, 2000, bf16))
    Pi = _as_i32(_rowmm(vertex_feat, Wi, 2000, bf16))
    Pk = _as_i32(_rowmm(edge_feat, Wk, 4000, bf16))
    Pa = _as_i32(_rowmm(angle_feat, Wa, 8000, bf16))

    x_i32 = _gather_sum(Pa, Pj, Pi, Pk, j_idx, i_idx, k_idx)
    x = jax.lax.bitcast_convert_type(x_i32, bf16).reshape(T, C2)

    stats = _stats(x, 8000)
    params = jnp.zeros((8, BOND), jnp.float32)
    params = params.at[0].set(bn_core_gamma).at[1].set(bn_core_beta)
    params = params.at[2].set(bn_gate_gamma).at[3].set(bn_gate_beta)
    u = _activate(x, stats, params, 4000)

    segsum = jax.ops.segment_sum(u, k_idx, num_segments=E)

    return _final_matmul(segsum, edge_feat, W_out)


# speedup vs baseline: 2.7329x; 2.7329x over previous
import functools

import jax
import jax.numpy as jnp
from jax import lax
from jax.experimental import pallas as pl
from jax.experimental.pallas import tpu as pltpu
from jax.experimental.pallas import tpu_sc as plsc

N, E, T = 10000, 160000, 320000
ATOM, BOND, ANGLE = 128, 128, 16
C2 = 2 * BOND

_SC_INFO = plsc.get_sparse_core_info()
_NC = _SC_INFO.num_cores
_NS = _SC_INFO.num_subcores
NW = _NC * _NS


_GB = 80
_CHUNK = T // NW
_NBLK = _CHUNK // _GB


_CW = C2 // 2


def _bf2(v):
    return plsc.bitcast(v, jnp.bfloat16)


def _gather_sum_body(pa_hbm, pj_hbm, pi_hbm, pk_hbm, j_hbm, i_hbm, k_hbm,
                     x_hbm, jb, ib, kb, xa, gj, gi, gk,
                     sem_a, sem_j, sem_i, sem_k):
    wid = lax.axis_index("s") * _NC + lax.axis_index("c")

    def blk_body(b, carry):
        base = wid * _CHUNK + b * _GB
        pltpu.sync_copy(j_hbm.at[pl.ds(base, _GB)], jb)
        pltpu.sync_copy(i_hbm.at[pl.ds(base, _GB)], ib)
        pltpu.sync_copy(k_hbm.at[pl.ds(base, _GB)], kb)
        ca = pltpu.async_copy(pa_hbm.at[pl.ds(base, _GB)], xa, sem_a)
        cj = pltpu.async_copy(pj_hbm.at[jb], gj, sem_j)
        ci = pltpu.async_copy(pi_hbm.at[ib], gi, sem_i)
        ck = pltpu.async_copy(pk_hbm.at[kb], gk, sem_k)
        ca.wait()
        cj.wait()
        ci.wait()
        ck.wait()

        def row_body(r, c2):
            for c in range(_CW // 16):
                sl = pl.ds(c * 16, 16)
                s = (_bf2(xa[r, sl]) + _bf2(gj[r, sl])) +                     (_bf2(gi[r, sl]) + _bf2(gk[r, sl]))
                xa[r, sl] = plsc.bitcast(s, jnp.int32)
            return c2

        lax.fori_loop(0, _GB, row_body, 0)
        pltpu.sync_copy(xa, x_hbm.at[pl.ds(base, _GB)])
        return carry

    lax.fori_loop(0, _NBLK, blk_body, 0)


def _gather_sum(Pa, Pj, Pi, Pk, j_idx, i_idx, k_idx):
    mesh = plsc.VectorSubcoreMesh(core_axis_name="c", subcore_axis_name="s")
    f = functools.partial(
        pl.kernel,
        mesh=mesh,
        compiler_params=pltpu.CompilerParams(needs_layout_passes=False),
        out_type=jax.ShapeDtypeStruct((T, _CW), jnp.int32),
        scratch_types=[
            pltpu.VMEM((_GB,), jnp.int32),
            pltpu.VMEM((_GB,), jnp.int32),
            pltpu.VMEM((_GB,), jnp.int32),
            pltpu.VMEM((_GB, _CW), jnp.int32),
            pltpu.VMEM((_GB, _CW), jnp.int32),
            pltpu.VMEM((_GB, _CW), jnp.int32),
            pltpu.VMEM((_GB, _CW), jnp.int32),
            pltpu.SemaphoreType.DMA,
            pltpu.SemaphoreType.DMA,
            pltpu.SemaphoreType.DMA,
            pltpu.SemaphoreType.DMA,
        ],
    )(_gather_sum_body)
    return f(Pa, Pj, Pi, Pk, j_idx, i_idx, k_idx)


def _pack_pair(core_f32, gate_f32):
    cb = jax.lax.bitcast_convert_type(core_f32, jnp.uint32)
    gb = jax.lax.bitcast_convert_type(gate_f32, jnp.uint32)
    cb = (cb + jnp.uint32(0x8000)) >> 16
    gb = (gb + jnp.uint32(0x8000)) & jnp.uint32(0xFFFF0000)
    return jax.lax.bitcast_convert_type(cb | gb, jnp.int32)


def _unpack_pair(word_i32):
    w = jax.lax.bitcast_convert_type(word_i32, jnp.uint32)
    core = jax.lax.bitcast_convert_type(w << 16, jnp.float32)
    gate = jax.lax.bitcast_convert_type(w & jnp.uint32(0xFFFF0000), jnp.float32)
    return core, gate


def _mm_body(x_ref, w_ref, o_ref):
    y = jnp.dot(x_ref[...], w_ref[...], preferred_element_type=jnp.float32)
    o_ref[...] = _pack_pair(y[:, :BOND], y[:, BOND:])


def _rowmm_packed(x, w, blk):
    m, k = x.shape
    n = w.shape[1]
    return pl.pallas_call(
        _mm_body,
        grid=(m // blk,),
        in_specs=[pl.BlockSpec((blk, k), lambda i: (i, 0)),
                  pl.BlockSpec((k, n), lambda i: (0, 0))],
        out_specs=pl.BlockSpec((blk, n // 2), lambda i: (i, 0)),
        out_shape=jax.ShapeDtypeStruct((m, n // 2), jnp.int32),
    )(x, w)


def _stats_body(x_ref, o_ref):
    @pl.when(pl.program_id(0) == 0)
    def _():
        o_ref[...] = jnp.zeros_like(o_ref)
    core, gate = _unpack_pair(x_ref[...])
    o_ref[0, :] += jnp.sum(core, axis=0)
    o_ref[1, :] += jnp.sum(core * core, axis=0)
    o_ref[2, :] += jnp.sum(gate, axis=0)
    o_ref[3, :] += jnp.sum(gate * gate, axis=0)


def _stats(x, blk):
    m, n = x.shape
    return pl.pallas_call(
        _stats_body,
        grid=(m // blk,),
        in_specs=[pl.BlockSpec((blk, n), lambda i: (i, 0))],
        out_specs=pl.BlockSpec((8, n), lambda i: (0, 0)),
        out_shape=jax.ShapeDtypeStruct((8, n), jnp.float32),
    )(x)


def _act_body(x_ref, st_ref, p_ref, o_ref):
    core, gate = _unpack_pair(x_ref[...])
    mean_c = st_ref[0, :] / T
    var_c = st_ref[1, :] / T - mean_c * mean_c
    inv_c = jax.lax.rsqrt(var_c + 1e-5)
    mean_g = st_ref[2, :] / T
    var_g = st_ref[3, :] / T - mean_g * mean_g
    inv_g = jax.lax.rsqrt(var_g + 1e-5)
    core = (core - mean_c) * inv_c * p_ref[0, :] + p_ref[1, :]
    gate = (gate - mean_g) * inv_g * p_ref[2, :] + p_ref[3, :]
    core = core * jax.nn.sigmoid(core)
    gate = jax.nn.sigmoid(gate)
    o_ref[...] = (core * gate).astype(o_ref.dtype)


def _activate(x, stats, params, blk):
    m = x.shape[0]
    return pl.pallas_call(
        _act_body,
        grid=(m // blk,),
        in_specs=[pl.BlockSpec((blk, BOND), lambda i: (i, 0)),
                  pl.BlockSpec((8, BOND), lambda i: (0, 0)),
                  pl.BlockSpec((8, BOND), lambda i: (0, 0))],
        out_specs=pl.BlockSpec((blk, BOND), lambda i: (i, 0)),
        out_shape=jax.ShapeDtypeStruct((m, BOND), jnp.bfloat16),
    )(x, stats, params)


def _final_body(seg_ref, edge_ref, w_ref, out_ref):
    out_ref[...] = jnp.dot(seg_ref[...], w_ref[...],
                           preferred_element_type=jnp.float32) + edge_ref[...]


def _final_matmul(segsum, edge_feat, W_out):
    BLK = 1600
    return pl.pallas_call(
        _final_body,
        grid=(E // BLK,),
        in_specs=[
            pl.BlockSpec((BLK, BOND), lambda i: (i, 0)),
            pl.BlockSpec((BLK, BOND), lambda i: (i, 0)),
            pl.BlockSpec((BOND, BOND), lambda i: (0, 0)),
        ],
        out_specs=pl.BlockSpec((BLK, BOND), lambda i: (i, 0)),
        out_shape=jax.ShapeDtypeStruct((E, BOND), jnp.float32),
    )(segsum, edge_feat, W_out)


def kernel(vertex_feat, edge_feat, angle_feat, edge_index, k_idx, j_idx, i_idx,
           W_core_src, W_core_dst, W_core_bond, W_core_angle,
           W_gate_src, W_gate_dst, W_gate_bond, W_gate_angle,
           bn_core_gamma, bn_core_beta, bn_gate_gamma, bn_gate_beta, W_out):
    k_idx = k_idx.astype(jnp.int32)
    j_idx = j_idx.astype(jnp.int32)
    i_idx = i_idx.astype(jnp.int32)

    Wj = jnp.concatenate([W_core_src, W_gate_src], axis=1)
    Wi = jnp.concatenate([W_core_dst, W_gate_dst], axis=1)
    Wk = jnp.concatenate([W_core_bond, W_gate_bond], axis=1)
    Wa = jnp.concatenate([W_core_angle, W_gate_angle], axis=1)
    Pj = _rowmm_packed(vertex_feat, Wj, 2000)
    Pi = _rowmm_packed(vertex_feat, Wi, 2000)
    Pk = _rowmm_packed(edge_feat, Wk, 4000)
    Pa = _rowmm_packed(angle_feat, Wa, 8000)

    x = _gather_sum(Pa, Pj, Pi, Pk, j_idx, i_idx, k_idx)

    stats = _stats(x, 8000)
    params = jnp.zeros((8, BOND), jnp.float32)
    params = params.at[0].set(bn_core_gamma).at[1].set(bn_core_beta)
    params = params.at[2].set(bn_gate_gamma).at[3].set(bn_gate_beta)
    u = _activate(x, stats, params, 4000)

    segsum = jax.ops.segment_sum(u, k_idx, num_segments=E)

    return _final_matmul(segsum, edge_feat, W_out)
